# baseline (speedup 1.0000x reference)
import functools

import jax
import jax.numpy as jnp
import numpy as np
from jax import lax
from jax.experimental import pallas as pl
from jax.experimental.pallas import tpu as pltpu
from jax.experimental.pallas import tpu_sc as plsc

N = 10000
E = 320000
F = 16
NC = 2
NS = 16
NW = NC * NS
EPW = E // NW
CH = 2000
NCH = EPW // CH
NPS = N // NS
NP = N // 8
EP = E // 8

_R_np = np.zeros((F, F * F), np.float32)
for _i in range(F):
    _R_np[_i, _i * F:(_i + 1) * F] = 1.0


def _unpack(x):
    b = x.shape[0]
    return jnp.concatenate([x[:, 16 * a:16 * (a + 1)] for a in range(8)], axis=0)


def _pack(m):
    b = m.shape[0] // 8
    return jnp.concatenate([m[a * b:(a + 1) * b, :] for a in range(8)], axis=1)


def _gather_body(h_hbm, src_hbm, out_hbm, idx_all, rows2, semg, semw2):
    c = lax.axis_index("c")
    s = lax.axis_index("s")
    w = c * NS + s
    base = w * EPW
    pltpu.sync_copy(src_hbm.at[pl.ds(base, EPW)], idx_all)
    writes = [None, None]
    for k in range(NCH):
        b = k % 2
        if writes[b] is not None:
            writes[b].wait()
        pltpu.async_copy(
            h_hbm.at[idx_all.at[pl.ds(k * CH, CH)]], rows2.at[b], semg).wait()
        writes[b] = pltpu.async_copy(
            rows2.at[b], out_hbm.at[w * NCH + k], semw2.at[b])
    for wcp in writes:
        if wcp is not None:
            wcp.wait()


@functools.lru_cache(maxsize=None)
def _sc_gather():
    return pl.kernel(
        _gather_body,
        out_type=jax.ShapeDtypeStruct((NW * NCH, CH, F), jnp.float32),
        mesh=plsc.VectorSubcoreMesh(core_axis_name="c", subcore_axis_name="s"),
        scratch_types=[
            pltpu.VMEM((EPW,), jnp.int32),
            pltpu.VMEM((2, CH, F), jnp.float32),
            pltpu.SemaphoreType.DMA,
            pltpu.SemaphoreType.DMA((2,)),
        ],
        compiler_params=pltpu.CompilerParams(use_tc_tiling_on_sc=False),
    )


def _scatter_body(m_hbm, dst_hbm, zeros_hbm, out_hbm, idx2, rows2, acc, semr2):
    c = lax.axis_index("c")
    s = lax.axis_index("s")
    pltpu.sync_copy(zeros_hbm.at[pl.ds(s * NPS, NPS)], acc.at[pl.ds(s * NPS, NPS)])
    w = c * NS + s
    base = w * EPW

    def start_read(k, b):
        pltpu.async_copy(dst_hbm.at[pl.ds(base + k * CH, CH)], idx2.at[b], semr2.at[b])
        pltpu.async_copy(m_hbm.at[w * NCH + k], rows2.at[b], semr2.at[b])

    def wait_read(b):
        pltpu.make_async_copy(
            dst_hbm.at[pl.ds(base, CH)], idx2.at[b], semr2.at[b]).wait()
        pltpu.make_async_copy(
            m_hbm.at[w * NCH], rows2.at[b], semr2.at[b]).wait()

    start_read(0, 0)
    plsc.subcore_barrier()
    for k in range(NCH):
        b = k % 2
        wait_read(b)
        if k + 1 < NCH:
            start_read(k + 1, 1 - b)
        pltpu.sync_copy(rows2.at[b], acc.at[idx2.at[b]], add=True)
    plsc.subcore_barrier()
    pltpu.sync_copy(acc.at[pl.ds(s * NPS, NPS)], out_hbm.at[c, pl.ds(s * NPS, NPS)])


@functools.lru_cache(maxsize=None)
def _sc_scatter():
    return pl.kernel(
        _scatter_body,
        out_type=jax.ShapeDtypeStruct((NC, N, F), jnp.float32),
        mesh=plsc.VectorSubcoreMesh(core_axis_name="c", subcore_axis_name="s"),
        scratch_types=[
            pltpu.VMEM((2, CH), jnp.int32),
            pltpu.VMEM((2, CH, F), jnp.float32),
            pltpu.VMEM_SHARED((N, F), jnp.float32),
            pltpu.SemaphoreType.DMA((2,)),
        ],
        compiler_params=pltpu.CompilerParams(use_tc_tiling_on_sc=False),
    )


BMP = 1000
BM = BMP * 8

_S_np = np.tile(np.eye(F, dtype=np.float32), (F, 1))


def _msg_math(y_h, y_ef, We, be, Rm, Sm):
    wf = jnp.dot(y_ef, We, preferred_element_type=jnp.float32) + be
    hexp = jnp.dot(y_h, Rm, preferred_element_type=jnp.float32)
    return jnp.dot(hexp * wf, Sm, preferred_element_type=jnp.float32)


def _msg1_body(hsrcp_ref, efraw_ref, We_ref, be_ref, R_ref, S_ref, mp_ref, efp_ref):
    y_ef = efraw_ref[...]
    m16 = _msg_math(_unpack(hsrcp_ref[...]), y_ef, We_ref[...], be_ref[...],
                    R_ref[...], S_ref[...])
    mp_ref[...] = _pack(m16)
    efp_ref[...] = _pack(y_ef)


def _msg2_body(hsrcp_ref, efp_ref, We_ref, be_ref, R_ref, S_ref, mp_ref):
    y_ef = _unpack(efp_ref[...])
    m16 = _msg_math(_unpack(hsrcp_ref[...]), y_ef, We_ref[...], be_ref[...],
                    R_ref[...], S_ref[...])
    mp_ref[...] = _pack(m16)


def _messages1(h_srcp, efeat, We, be2, Rm, Sm):
    return pl.pallas_call(
        _msg1_body,
        grid=(EP // BMP,),
        in_specs=[
            pl.BlockSpec((BMP, 128), lambda i: (i, 0)),
            pl.BlockSpec((BM, F), lambda i: (i, 0)),
            pl.BlockSpec((F, F * F), lambda i: (0, 0)),
            pl.BlockSpec((1, F * F), lambda i: (0, 0)),
            pl.BlockSpec((F, F * F), lambda i: (0, 0)),
            pl.BlockSpec((F * F, F), lambda i: (0, 0)),
        ],
        out_specs=[
            pl.BlockSpec((BMP, 128), lambda i: (i, 0)),
            pl.BlockSpec((BMP, 128), lambda i: (i, 0)),
        ],
        out_shape=[
            jax.ShapeDtypeStruct((EP, 128), jnp.float32),
            jax.ShapeDtypeStruct((EP, 128), jnp.float32),
        ],
    )(h_srcp, efeat, We, be2, Rm, Sm)


def _messages2(h_srcp, efeat_p, We, be2, Rm, Sm):
    return pl.pallas_call(
        _msg2_body,
        grid=(EP // BMP,),
        in_specs=[
            pl.BlockSpec((BMP, 128), lambda i: (i, 0)),
            pl.BlockSpec((BMP, 128), lambda i: (i, 0)),
            pl.BlockSpec((F, F * F), lambda i: (0, 0)),
            pl.BlockSpec((1, F * F), lambda i: (0, 0)),
            pl.BlockSpec((F, F * F), lambda i: (0, 0)),
            pl.BlockSpec((F * F, F), lambda i: (0, 0)),
        ],
        out_specs=pl.BlockSpec((BMP, 128), lambda i: (i, 0)),
        out_shape=jax.ShapeDtypeStruct((EP, 128), jnp.float32),
    )(h_srcp, efeat_p, We, be2, Rm, Sm)


def _gru_body(rst2p_ref, hp_ref, WihT_ref, WhhT_ref, bih_ref, bhh_ref, outp_ref):
    rst_p = rst2p_ref[0:NP, :] + rst2p_ref[NP:2 * NP, :]
    rst = _unpack(rst_p)
    h = _unpack(hp_ref[...])
    gi = jnp.dot(rst, WihT_ref[...], preferred_element_type=jnp.float32) + bih_ref[...]
    gh = jnp.dot(h, WhhT_ref[...], preferred_element_type=jnp.float32) + bhh_ref[...]
    r = jax.nn.sigmoid(gi[:, 0:F] + gh[:, 0:F])
    z = jax.nn.sigmoid(gi[:, F:2 * F] + gh[:, F:2 * F])
    n = jnp.tanh(gi[:, 2 * F:3 * F] + r * gh[:, 2 * F:3 * F])
    outp_ref[...] = _pack((1.0 - z) * n + z * h)


def _gru(rst2_p, h_p, WihT, WhhT, bih2, bhh2):
    return pl.pallas_call(
        _gru_body,
        out_shape=jax.ShapeDtypeStruct((NP, 128), jnp.float32),
    )(rst2_p, h_p, WihT, WhhT, bih2, bhh2)


def kernel(feat, edge_index, efeat, W_edge, b_edge, W_ih, W_hh, b_ih, b_hh):
    Rm = jnp.asarray(_R_np)
    Sm = jnp.asarray(_S_np)
    be2 = b_edge.reshape(1, F * F)
    WihT = W_ih.T
    WhhT = W_hh.T
    bih2 = b_ih.reshape(1, 3 * F)
    bhh2 = b_hh.reshape(1, 3 * F)
    zeros_nf = jnp.zeros((N, F), jnp.float32)

    def _eperm(v):
        return v.reshape(EP // BMP, 8, BMP).transpose(0, 2, 1).reshape(E)

    src_p = _eperm(edge_index[0])
    dst_p = _eperm(edge_index[1])

    h_p = feat.reshape(NP, 128)
    efeat_p = None
    for step in range(2):
        h_lin = h_p.reshape(N, F)
        h_src3 = _sc_gather()(h_lin, src_p)
        h_srcp = h_src3.reshape(EP, 128)
        if step == 0:
            m_p, efeat_p = _messages1(h_srcp, efeat, W_edge, be2, Rm, Sm)
        else:
            m_p = _messages2(h_srcp, efeat_p, W_edge, be2, Rm, Sm)
        m3 = m_p.reshape(NW * NCH, CH, F)
        rst2 = _sc_scatter()(m3, dst_p, zeros_nf)
        rst2_p = rst2.reshape(2 * NP, 128)
        h_p = _gru(rst2_p, h_p, WihT, WhhT, bih2, bhh2)
    return h_p.reshape(N, F)
